# baseline (speedup 1.0000x reference)
import functools

import numpy as np
import jax
import jax.numpy as jnp
from jax import lax
from jax.experimental import pallas as pl
from jax.experimental.pallas import tpu as pltpu
from jax.experimental.pallas import tpu_sc as plsc

_N = 1048576
_MESH = 2000
_W = 2 * _MESH + 1
_NWIN = 500
_NW = 32
_PPW = _N // _NW
_C = 1024
_NCHUNK = _PPW // _C
_L = 16

_DELTA = np.float32(1.0) / np.float32(_MESH)


def _basis(l):
    b0 = (l - 1.0) * (l - 2.0) * 0.5
    b1 = l * (2.0 - l)
    b2 = l * (l - 1.0) * 0.5
    return b0, b1, b2


def _cell(t):
    ei = jnp.minimum(t.astype(jnp.int32), _MESH - 1)
    loc = (t - ei.astype(jnp.float32)) * 2.0
    return ei, loc


def _one_set():
    return ([pltpu.VMEM((_C,), jnp.float32),
             pltpu.VMEM((_C,), jnp.float32)]
            + [pltpu.VMEM((_C,), jnp.int32) for _ in range(3)]
            + [pltpu.VMEM((_C, _L), jnp.float32) for _ in range(3)]
            + [pltpu.VMEM((_C,), jnp.float32),
               pltpu.SemaphoreType.DMA])


@jax.jit
def _sc_interp(x, y, wtab):
    mesh = plsc.VectorSubcoreMesh(core_axis_name="c", subcore_axis_name="s")

    @functools.partial(
        pl.kernel,
        mesh=mesh,
        out_type=jax.ShapeDtypeStruct((_N,), jnp.float32),
        scratch_types=_one_set() + _one_set(),
        compiler_params=pltpu.CompilerParams(needs_layout_passes=False,
                                             use_tc_tiling_on_sc=False),
    )
    def k(x_hbm, y_hbm, tab_hbm, out_hbm, *rest):
        sets = []
        for b in range(2):
            r = rest[b * 10:(b + 1) * 10]
            sets.append(dict(xv=r[0], yv=r[1], idxv=r[2:5], wv=r[5:8],
                             ov=r[8], sem=r[9]))
        wid = lax.axis_index("s") * 2 + lax.axis_index("c")
        base = wid * _PPW

        def load_and_fire(c, S):
            off = base + c * _C
            pltpu.sync_copy(x_hbm.at[pl.ds(off, _C)], S["xv"])
            pltpu.sync_copy(y_hbm.at[pl.ds(off, _C)], S["yv"])

            def build(i, carry2):
                s = i * _L
                ex, _ = _cell(S["xv"][pl.ds(s, _L)] / _DELTA)
                ey, _ = _cell(S["yv"][pl.ds(s, _L)] / _DELTA)
                win = (ex * 2) * _NWIN + (ey >> 2)
                for r in range(3):
                    S["idxv"][r][pl.ds(s, _L)] = win + r * _NWIN
                return carry2

            lax.fori_loop(0, _C // _L, build, 0, unroll=False)
            return [
                pltpu.async_copy(tab_hbm.at[S["idxv"][r]], S["wv"][r],
                                 S["sem"])
                for r in range(3)
            ]

        def drain_and_accum(c, S, copies):
            for cp in copies:
                cp.wait()

            def accum(i, carry2):
                s = i * _L
                iv = lax.iota(jnp.int32, _L) + s
                _, lx = _cell(S["xv"][pl.ds(s, _L)] / _DELTA)
                ey, ly = _cell(S["yv"][pl.ds(s, _L)] / _DELTA)
                d = (ey & 3) * 2
                bx = _basis(lx)
                by = _basis(ly)
                acc = None
                for r in range(3):
                    c0 = plsc.load_gather(S["wv"][r], [iv, d])
                    c1 = plsc.load_gather(S["wv"][r], [iv, d + 1])
                    c2 = plsc.load_gather(S["wv"][r], [iv, d + 2])
                    rs = c0 * by[0] + c1 * by[1] + c2 * by[2]
                    acc = rs * bx[r] if acc is None else acc + rs * bx[r]
                S["ov"][pl.ds(s, _L)] = acc
                return carry2

            lax.fori_loop(0, _C // _L, accum, 0, unroll=False)
            off = base + c * _C
            pltpu.sync_copy(S["ov"], out_hbm.at[pl.ds(off, _C)])

        pending = load_and_fire(0, sets[0])
        for c in range(1, _NCHUNK):
            nxt = load_and_fire(c, sets[c % 2])
            drain_and_accum(c - 1, sets[(c - 1) % 2], pending)
            pending = nxt
        drain_and_accum(_NCHUNK - 1, sets[(_NCHUNK - 1) % 2], pending)

    return k(x, y, wtab)


def kernel(inputs, interp_coe):
    x = inputs[:, 0]
    y = inputs[:, 1]
    p = jnp.pad(interp_coe, ((0, 0), (0, 15)))
    even = p.reshape(_W, 251, 16)
    odd = p[:, 8:4008].reshape(_W, 250, 16)
    wtab = jnp.stack([even[:, :250], odd], axis=2).reshape(_W * _NWIN, 16)
    return _sc_interp(x, y, wtab)


# speedup vs baseline: 1.5167x; 1.5167x over previous
import functools

import numpy as np
import jax
import jax.numpy as jnp
from jax import lax
from jax.experimental import pallas as pl
from jax.experimental.pallas import tpu as pltpu
from jax.experimental.pallas import tpu_sc as plsc

_N = 1048576
_MESH = 2000
_W = 2 * _MESH + 1
_NWIN = 500
_NW = 32
_PPW = _N // _NW
_C = 1024
_NCHUNK = _PPW // _C
_L = 16

_DELTA = np.float32(1.0) / np.float32(_MESH)


def _basis(l):
    b0 = (l - 1.0) * (l - 2.0) * 0.5
    b1 = l * (2.0 - l)
    b2 = l * (l - 1.0) * 0.5
    return b0, b1, b2


def _cell(t):
    ei = jnp.minimum(t.astype(jnp.int32), _MESH - 1)
    loc = (t - ei.astype(jnp.float32)) * 2.0
    return ei, loc


def _one_set():
    return ([pltpu.VMEM((_C,), jnp.float32),
             pltpu.VMEM((_C,), jnp.float32)]
            + [pltpu.VMEM((_C,), jnp.int32) for _ in range(3)]
            + [pltpu.VMEM((_C, _L), jnp.float32) for _ in range(3)]
            + [pltpu.VMEM((_C,), jnp.float32),
               pltpu.SemaphoreType.DMA])


@jax.jit
def _sc_interp(x, y, wtab):
    mesh = plsc.VectorSubcoreMesh(core_axis_name="c", subcore_axis_name="s")

    @functools.partial(
        pl.kernel,
        mesh=mesh,
        out_type=jax.ShapeDtypeStruct((_N,), jnp.float32),
        scratch_types=_one_set() + _one_set(),
        compiler_params=pltpu.CompilerParams(needs_layout_passes=False,
                                             use_tc_tiling_on_sc=False),
    )
    def k(x_hbm, y_hbm, tab_hbm, out_hbm, *rest):
        sets = []
        for b in range(2):
            r = rest[b * 10:(b + 1) * 10]
            sets.append(dict(xv=r[0], yv=r[1], idxv=r[2:5], wv=r[5:8],
                             ov=r[8], sem=r[9]))
        wid = lax.axis_index("s") * 2 + lax.axis_index("c")
        base = wid * _PPW

        def load_and_fire(c, S):
            off = base + c * _C
            pltpu.sync_copy(x_hbm.at[pl.ds(off, _C)], S["xv"])
            pltpu.sync_copy(y_hbm.at[pl.ds(off, _C)], S["yv"])

            def build(i, carry2):
                s = i * _L
                ex, _ = _cell(S["xv"][pl.ds(s, _L)] / _DELTA)
                ey, _ = _cell(S["yv"][pl.ds(s, _L)] / _DELTA)
                jj = ey >> 2
                m = jj >> 1
                odd = jj & 1
                row0 = ex * 2
                for r in range(3):
                    row = row0 + r
                    ie = row * 251 + m
                    io = _W * 251 + row * 250 + m
                    S["idxv"][r][pl.ds(s, _L)] = jnp.where(odd == 1, io, ie)
                return carry2

            lax.fori_loop(0, _C // _L, build, 0, unroll=False)
            return [
                pltpu.async_copy(tab_hbm.at[S["idxv"][r]], S["wv"][r],
                                 S["sem"])
                for r in range(3)
            ]

        def drain_and_accum(c, S, copies):
            for cp in copies:
                cp.wait()

            def accum(i, carry2):
                s = i * _L
                iv = lax.iota(jnp.int32, _L) + s
                _, lx = _cell(S["xv"][pl.ds(s, _L)] / _DELTA)
                ey, ly = _cell(S["yv"][pl.ds(s, _L)] / _DELTA)
                d = (ey & 3) * 2
                bx = _basis(lx)
                by = _basis(ly)
                acc = None
                for r in range(3):
                    c0 = plsc.load_gather(S["wv"][r], [iv, d])
                    c1 = plsc.load_gather(S["wv"][r], [iv, d + 1])
                    c2 = plsc.load_gather(S["wv"][r], [iv, d + 2])
                    rs = c0 * by[0] + c1 * by[1] + c2 * by[2]
                    acc = rs * bx[r] if acc is None else acc + rs * bx[r]
                S["ov"][pl.ds(s, _L)] = acc
                return carry2

            lax.fori_loop(0, _C // _L, accum, 0, unroll=False)
            off = base + c * _C
            pltpu.sync_copy(S["ov"], out_hbm.at[pl.ds(off, _C)])

        pending = load_and_fire(0, sets[0])
        for c in range(1, _NCHUNK):
            nxt = load_and_fire(c, sets[c % 2])
            drain_and_accum(c - 1, sets[(c - 1) % 2], pending)
            pending = nxt
        drain_and_accum(_NCHUNK - 1, sets[(_NCHUNK - 1) % 2], pending)

    return k(x, y, wtab)


def kernel(inputs, interp_coe):
    x = inputs[:, 0]
    y = inputs[:, 1]
    p = jnp.pad(interp_coe, ((0, 0), (0, 15)))
    even = p.reshape(_W * 251, 16)
    odd = p[:, 8:4008].reshape(_W * 250, 16)
    wtab = jnp.concatenate([even, odd], axis=0)
    return _sc_interp(x, y, wtab)


# speedup vs baseline: 5.3643x; 3.5369x over previous
import functools

import numpy as np
import jax
import jax.numpy as jnp
from jax import lax
from jax.experimental import pallas as pl
from jax.experimental.pallas import tpu as pltpu
from jax.experimental.pallas import tpu_sc as plsc

_N = 1048576
_MESH = 2000
_W = 2 * _MESH + 1
_NWIN = 500
_NW = 32
_PPW = _N // _NW
_C = 1024
_NCHUNK = _PPW // _C
_L = 16

_DELTA = np.float32(1.0) / np.float32(_MESH)


def _basis(l):
    b0 = (l - 1.0) * (l - 2.0) * 0.5
    b1 = l * (2.0 - l)
    b2 = l * (l - 1.0) * 0.5
    return b0, b1, b2


def _cell(t):
    ei = jnp.minimum(t.astype(jnp.int32), _MESH - 1)
    loc = (t - ei.astype(jnp.float32)) * 2.0
    return ei, loc


def _one_set():
    return ([pltpu.VMEM((_C,), jnp.float32),
             pltpu.VMEM((_C,), jnp.float32)]
            + [pltpu.VMEM((_C,), jnp.int32) for _ in range(3)]
            + [pltpu.VMEM((_C, _L), jnp.float32) for _ in range(3)]
            + [pltpu.VMEM((_C,), jnp.float32),
               pltpu.SemaphoreType.DMA])


@jax.jit
def _sc_interp(x, y, wtab):
    mesh = plsc.VectorSubcoreMesh(core_axis_name="c", subcore_axis_name="s")

    @functools.partial(
        pl.kernel,
        mesh=mesh,
        out_type=jax.ShapeDtypeStruct((_N,), jnp.float32),
        scratch_types=_one_set() + _one_set(),
        compiler_params=pltpu.CompilerParams(needs_layout_passes=False,
                                             use_tc_tiling_on_sc=False),
    )
    def k(x_hbm, y_hbm, tab_hbm, out_hbm, *rest):
        sets = []
        for b in range(2):
            r = rest[b * 10:(b + 1) * 10]
            sets.append(dict(xv=r[0], yv=r[1], idxv=r[2:5], wv=r[5:8],
                             ov=r[8], sem=r[9]))
        wid = lax.axis_index("s") * 2 + lax.axis_index("c")
        base = wid * _PPW

        def load_and_fire(c, S):
            off = base + c * _C
            pltpu.sync_copy(x_hbm.at[pl.ds(off, _C)], S["xv"])
            pltpu.sync_copy(y_hbm.at[pl.ds(off, _C)], S["yv"])

            def build(i, carry2):
                s = i * _L
                ex, _ = _cell(S["xv"][pl.ds(s, _L)] / _DELTA)
                ey, _ = _cell(S["yv"][pl.ds(s, _L)] / _DELTA)
                jj = ey >> 2
                moff = (jj >> 1) + (jj & 1) * 251
                row0 = ex * 2
                for r in range(3):
                    S["idxv"][r][pl.ds(s, _L)] = (row0 + r) * 501 + moff
                return carry2

            lax.fori_loop(0, _C // _L, build, 0, unroll=False)
            return [
                pltpu.async_copy(tab_hbm.at[S["idxv"][r]], S["wv"][r],
                                 S["sem"])
                for r in range(3)
            ]

        def drain_and_accum(c, S, copies):
            for cp in copies:
                cp.wait()

            def accum(i, carry2):
                s = i * _L
                iv = lax.iota(jnp.int32, _L) + s
                _, lx = _cell(S["xv"][pl.ds(s, _L)] / _DELTA)
                ey, ly = _cell(S["yv"][pl.ds(s, _L)] / _DELTA)
                d = (ey & 3) * 2
                bx = _basis(lx)
                by = _basis(ly)
                acc = None
                for r in range(3):
                    c0 = plsc.load_gather(S["wv"][r], [iv, d])
                    c1 = plsc.load_gather(S["wv"][r], [iv, d + 1])
                    c2 = plsc.load_gather(S["wv"][r], [iv, d + 2])
                    rs = c0 * by[0] + c1 * by[1] + c2 * by[2]
                    acc = rs * bx[r] if acc is None else acc + rs * bx[r]
                S["ov"][pl.ds(s, _L)] = acc
                return carry2

            lax.fori_loop(0, _C // _L, accum, 0, unroll=False)
            off = base + c * _C
            pltpu.sync_copy(S["ov"], out_hbm.at[pl.ds(off, _C)])

        pending = load_and_fire(0, sets[0])
        for c in range(1, _NCHUNK):
            nxt = load_and_fire(c, sets[c % 2])
            drain_and_accum(c - 1, sets[(c - 1) % 2], pending)
            pending = nxt
        drain_and_accum(_NCHUNK - 1, sets[(_NCHUNK - 1) % 2], pending)

    return k(x, y, wtab)


_RB = 8
_NPROG = (_W + _RB - 1) // _RB


def _relayout_body(i_ref, o_ref):
    rows = i_ref[...]
    o_ref[:, 0:_W] = rows
    o_ref[:, _W:4016] = jnp.zeros((_RB, 4016 - _W), jnp.float32)
    o_ref[:, 4016:8009] = rows[:, 8:_W]
    o_ref[:, 8009:8016] = jnp.zeros((_RB, 7), jnp.float32)


@jax.jit
def _relayout(coe):
    wide = pl.pallas_call(
        _relayout_body,
        grid=(_NPROG,),
        in_specs=[pl.BlockSpec((_RB, _W), lambda i: (i, 0))],
        out_specs=pl.BlockSpec((_RB, 8016), lambda i: (i, 0)),
        out_shape=jax.ShapeDtypeStruct((_NPROG * _RB, 8016), jnp.float32),
    )(coe)
    return wide.reshape(_NPROG * _RB * 501, 16)


def kernel(inputs, interp_coe):
    x = inputs[:, 0]
    y = inputs[:, 1]
    return _sc_interp(x, y, _relayout(interp_coe))


# speedup vs baseline: 12.2352x; 2.2808x over previous
import functools

import numpy as np
import jax
import jax.numpy as jnp
from jax import lax
from jax.experimental import pallas as pl
from jax.experimental.pallas import tpu as pltpu
from jax.experimental.pallas import tpu_sc as plsc

_N = 1048576
_MESH = 2000
_W = 2 * _MESH + 1
_NWIN = 500
_NW = 32
_PPW = _N // _NW
_C = 1024
_NCHUNK = _PPW // _C
_L = 16

_DELTA = np.float32(1.0) / np.float32(_MESH)


def _basis(l):
    b0 = (l - 1.0) * (l - 2.0) * 0.5
    b1 = l * (2.0 - l)
    b2 = l * (l - 1.0) * 0.5
    return b0, b1, b2


def _cell(t):
    ei = jnp.minimum(t.astype(jnp.int32), _MESH - 1)
    loc = (t - ei.astype(jnp.float32)) * 2.0
    return ei, loc


def _one_set():
    return ([pltpu.VMEM((_C,), jnp.float32),
             pltpu.VMEM((_C,), jnp.float32)]
            + [pltpu.VMEM((_C,), jnp.int32) for _ in range(3)]
            + [pltpu.VMEM((_C, _L), jnp.float32) for _ in range(3)]
            + [pltpu.VMEM((_C,), jnp.float32),
               pltpu.SemaphoreType.DMA])


@jax.jit
def _sc_interp(x, y, wtab):
    mesh = plsc.VectorSubcoreMesh(core_axis_name="c", subcore_axis_name="s")

    @functools.partial(
        pl.kernel,
        mesh=mesh,
        out_type=jax.ShapeDtypeStruct((_N,), jnp.float32),
        scratch_types=_one_set() + _one_set(),
        compiler_params=pltpu.CompilerParams(needs_layout_passes=False,
                                             use_tc_tiling_on_sc=False),
    )
    def k(x_hbm, y_hbm, tab_hbm, out_hbm, *rest):
        sets = []
        for b in range(2):
            r = rest[b * 10:(b + 1) * 10]
            sets.append(dict(xv=r[0], yv=r[1], idxv=r[2:5], wv=r[5:8],
                             ov=r[8], sem=r[9]))
        wid = lax.axis_index("s") * 2 + lax.axis_index("c")
        base = wid * _PPW

        def load_and_fire(c, S):
            off = base + c * _C
            pltpu.sync_copy(x_hbm.at[pl.ds(off, _C)], S["xv"])
            pltpu.sync_copy(y_hbm.at[pl.ds(off, _C)], S["yv"])

            def build(i, carry2):
                s = i * _L
                ex, _ = _cell(S["xv"][pl.ds(s, _L)] / _DELTA)
                ey, _ = _cell(S["yv"][pl.ds(s, _L)] / _DELTA)
                jj = ey >> 2
                moff = (jj >> 1) + (jj & 1) * 251
                row0 = ex * 2
                for r in range(3):
                    S["idxv"][r][pl.ds(s, _L)] = (row0 + r) * 501 + moff
                return carry2

            lax.fori_loop(0, _C // _L, build, 0, unroll=False)
            return [
                pltpu.async_copy(tab_hbm.at[S["idxv"][r]], S["wv"][r],
                                 S["sem"])
                for r in range(3)
            ]

        def drain_and_accum(c, S, copies):
            for cp in copies:
                cp.wait()

            def accum(i, carry2):
                s = i * _L
                iv = lax.iota(jnp.int32, _L) + s
                _, lx = _cell(S["xv"][pl.ds(s, _L)] / _DELTA)
                ey, ly = _cell(S["yv"][pl.ds(s, _L)] / _DELTA)
                d = (ey & 3) * 2
                bx = _basis(lx)
                by = _basis(ly)
                acc = None
                for r in range(3):
                    c0 = plsc.load_gather(S["wv"][r], [iv, d])
                    c1 = plsc.load_gather(S["wv"][r], [iv, d + 1])
                    c2 = plsc.load_gather(S["wv"][r], [iv, d + 2])
                    rs = c0 * by[0] + c1 * by[1] + c2 * by[2]
                    acc = rs * bx[r] if acc is None else acc + rs * bx[r]
                S["ov"][pl.ds(s, _L)] = acc
                return carry2

            lax.fori_loop(0, _C // _L, accum, 0, unroll=False)
            off = base + c * _C
            pltpu.sync_copy(S["ov"], out_hbm.at[pl.ds(off, _C)])

        pending = load_and_fire(0, sets[0])
        for c in range(1, _NCHUNK):
            nxt = load_and_fire(c, sets[c % 2])
            drain_and_accum(c - 1, sets[(c - 1) % 2], pending)
            pending = nxt
        drain_and_accum(_NCHUNK - 1, sets[(_NCHUNK - 1) % 2], pending)

    return k(x, y, wtab)


_RB = 64
_NPROG = (_W + _RB - 1) // _RB


def _relayout_body(i_ref, o_ref):
    for rr in range(_RB):
        o_ref[pl.ds(rr * 8016, _W)] = i_ref[rr]
        o_ref[pl.ds(rr * 8016 + 4016, _W - 8)] = i_ref[rr, 8:_W]


@jax.jit
def _relayout(coe):
    wide = pl.pallas_call(
        _relayout_body,
        grid=(_NPROG,),
        in_specs=[pl.BlockSpec((_RB, _W), lambda i: (i, 0))],
        out_specs=pl.BlockSpec((_RB * 8016,), lambda i: (i,)),
        out_shape=jax.ShapeDtypeStruct((_NPROG * _RB * 8016,), jnp.float32),
    )(coe)
    return wide.reshape(_NPROG * _RB * 501, 16)


def kernel(inputs, interp_coe):
    x = inputs[:, 0]
    y = inputs[:, 1]
    return _sc_interp(x, y, _relayout(interp_coe))
